# baseline (speedup 1.0000x reference)
import functools

import jax
import jax.numpy as jnp
from jax import lax
from jax.experimental import pallas as pl
from jax.experimental.pallas import tpu as pltpu
from jax.experimental.pallas import tpu_sc as plsc

NC, NS, L = 2, 16, 16
NW = NC * NS
N = 10000
E = 320000
NPAD = 10240
EPW = E // NW
LCAP = NPAD
HEAD = 512
RPW = NPAD // NW
LB = 8
FLUSH = 2048
PCAP = FLUSH + HEAD + L

_mesh = plsc.VectorSubcoreMesh(core_axis_name="c", subcore_axis_name="s")


@functools.partial(
    pl.kernel,
    out_type=(jax.ShapeDtypeStruct((NW, NPAD), jnp.float32),
              jax.ShapeDtypeStruct((NW, NPAD), jnp.float32)),
    mesh=_mesh,
    scratch_types=[pltpu.VMEM((EPW,), jnp.int32),
                   pltpu.VMEM((EPW,), jnp.int32),
                   pltpu.VMEM((NPAD,), jnp.float32),
                   pltpu.VMEM((NPAD,), jnp.float32),
                   pltpu.SemaphoreType.DMA],
    compiler_params=pltpu.CompilerParams(needs_layout_passes=False),
)
def _hist(src_hbm, dst_hbm, degp_hbm, c0p_hbm, srcv, dstv, hdeg, hc0, sem):
    c = lax.axis_index("c")
    s = lax.axis_index("s")
    w = s * NC + c
    d1 = pltpu.async_copy(src_hbm.at[pl.ds(w * EPW, EPW)], srcv, sem)
    d2 = pltpu.async_copy(dst_hbm.at[pl.ds(w * EPW, EPW)], dstv, sem)
    zero = jnp.zeros((L,), jnp.float32)

    def zbody(i, carry):
        hdeg[pl.ds(i * L, L)] = zero
        hc0[pl.ds(i * L, L)] = zero
        return carry

    lax.fori_loop(0, NPAD // L, zbody, 0)
    ones = jnp.ones((L,), jnp.float32)
    d1.wait()
    d2.wait()

    def body(i, carry):
        d16 = dstv[pl.ds(i * L, L)]
        s16 = srcv[pl.ds(i * L, L)]
        plsc.addupdate_scatter(hdeg, [d16], ones)
        plsc.addupdate_scatter(hc0, [s16], ones, mask=d16 == 0)
        return carry

    lax.fori_loop(0, EPW // L, body, 0)
    d3 = pltpu.async_copy(hdeg, degp_hbm.at[w], sem)
    d4 = pltpu.async_copy(hc0, c0p_hbm.at[w], sem)
    d3.wait()
    d4.wait()


def _mm_body(x_ref, w_ref, degp_ref, c0p_ref, deg0_ref,
             hs_ref, dinv_ref, a0_ref, needed_ref):
    mblk = NPAD // 8
    deg = jnp.sum(degp_ref[...], axis=0, keepdims=True) + 1.0
    dinv = lax.rsqrt(deg)
    cnt0 = jnp.sum(c0p_ref[...], axis=0, keepdims=True)
    deg0 = jnp.sum(deg0_ref[...], axis=0, keepdims=True)[0, 0] + 1.0
    dinv0 = lax.rsqrt(deg0)
    col = (lax.broadcasted_iota(jnp.int32, (1, mblk), 1)
           + pl.program_id(0) * mblk)
    a0 = dinv0 * dinv * cnt0 + jnp.where(col == 0, dinv0 * dinv0, 0.0)
    needed = jnp.where((cnt0 > 0.0) | (col == 0), 1.0, 0.0)
    dinv_ref[...] = dinv
    a0_ref[...] = a0
    needed_ref[...] = needed
    h = jnp.dot(x_ref[...], w_ref[...], preferred_element_type=jnp.float32)
    hs_ref[...] = h * dinv[0][:, None]


def _mm(xp, W1, degp, c0p):
    mblk = NPAD // 8
    return pl.pallas_call(
        _mm_body,
        grid=(8,),
        in_specs=[
            pl.BlockSpec((mblk, xp.shape[1]), lambda g: (g, 0)),
            pl.BlockSpec(W1.shape, lambda g: (0, 0)),
            pl.BlockSpec((NW, mblk), lambda g: (0, g)),
            pl.BlockSpec((NW, mblk), lambda g: (0, g)),
            pl.BlockSpec((NW, 128), lambda g: (0, 0)),
        ],
        out_specs=(
            pl.BlockSpec((mblk, W1.shape[1]), lambda g: (g, 0)),
            pl.BlockSpec((1, mblk), lambda g: (0, g)),
            pl.BlockSpec((1, mblk), lambda g: (0, g)),
            pl.BlockSpec((1, mblk), lambda g: (0, g)),
        ),
        out_shape=(
            jax.ShapeDtypeStruct((NPAD, W1.shape[1]), jnp.float32),
            jax.ShapeDtypeStruct((1, NPAD), jnp.float32),
            jax.ShapeDtypeStruct((1, NPAD), jnp.float32),
            jax.ShapeDtypeStruct((1, NPAD), jnp.float32),
        ),
    )(xp, W1, degp, c0p, degp)


@functools.partial(
    pl.kernel,
    out_type=(jax.ShapeDtypeStruct((NW, 2, HEAD), jnp.int32),
              jax.ShapeDtypeStruct((NW, LCAP), jnp.int32),
              jax.ShapeDtypeStruct((NW, LCAP), jnp.int32),
              jax.ShapeDtypeStruct((NW, L), jnp.int32)),
    mesh=_mesh,
    scratch_types=[pltpu.VMEM((EPW,), jnp.int32),
                   pltpu.VMEM((EPW,), jnp.int32),
                   pltpu.VMEM((NPAD,), jnp.float32),
                   pltpu.VMEM((LCAP,), jnp.int32),
                   pltpu.VMEM((LCAP,), jnp.int32),
                   pltpu.VMEM((L,), jnp.int32),
                   pltpu.SemaphoreType.DMA],
    compiler_params=pltpu.CompilerParams(needs_layout_passes=False),
)
def _filt(src_hbm, dst_hbm, needed_hbm, heads_hbm, ls_hbm, ld_hbm, cnt_hbm,
          srcv, dstv, neededv, psrc, pdst, kv, sem):
    c = lax.axis_index("c")
    s = lax.axis_index("s")
    w = s * NC + c
    d1 = pltpu.async_copy(src_hbm.at[pl.ds(w * EPW, EPW)], srcv, sem)
    d2 = pltpu.async_copy(dst_hbm.at[pl.ds(w * EPW, EPW)], dstv, sem)
    d3 = pltpu.async_copy(needed_hbm, neededv, sem)

    zi = jnp.zeros((L,), jnp.int32)
    dummy = jnp.full((L,), NPAD, jnp.int32)

    def pf(i, carry):
        psrc[pl.ds(i * L, L)] = zi
        pdst[pl.ds(i * L, L)] = dummy
        return carry

    lax.fori_loop(0, LCAP // L, pf, 0)
    d1.wait()
    d2.wait()
    d3.wait()

    def comp(i, k):
        d16 = dstv[pl.ds(i * L, L)]
        s16 = srcv[pl.ds(i * L, L)]
        nd = plsc.load_gather(neededv, [d16])
        m = nd > 0.0
        plsc.store_compressed(psrc.at[pl.ds(k, L)], s16, mask=m)
        plsc.store_compressed(pdst.at[pl.ds(k, L)], d16, mask=m)
        pc = plsc.all_reduce_population_count(m)
        return k + pc[0]

    k = lax.fori_loop(0, EPW // L, comp, jnp.int32(0))

    kv[pl.ds(0, L)] = jnp.broadcast_to(k, (L,)).astype(jnp.int32)
    d4 = pltpu.async_copy(psrc.at[pl.ds(0, HEAD)], heads_hbm.at[w, 0], sem)
    d5 = pltpu.async_copy(pdst.at[pl.ds(0, HEAD)], heads_hbm.at[w, 1], sem)
    d6 = pltpu.async_copy(kv, cnt_hbm.at[w], sem)

    @pl.when(k > HEAD)
    def _():
        pltpu.sync_copy(psrc, ls_hbm.at[w])
        pltpu.sync_copy(pdst, ld_hbm.at[w])

    d4.wait()
    d5.wait()
    d6.wait()


def _make_agg(H):
    @functools.partial(
        pl.kernel,
        out_type=jax.ShapeDtypeStruct((NW, H), jnp.float32),
        mesh=_mesh,
        scratch_types=[pltpu.VMEM((RPW, H), jnp.float32),
                       pltpu.VMEM((NW, L), jnp.int32),
                       pltpu.VMEM((NW, 2, HEAD), jnp.int32),
                       pltpu.VMEM((HEAD,), jnp.int32),
                       pltpu.VMEM((HEAD,), jnp.int32),
                       pltpu.VMEM((PCAP,), jnp.int32),
                       pltpu.VMEM((PCAP,), jnp.int32),
                       pltpu.VMEM((L, H), jnp.float32),
                       pltpu.VMEM((RPW + L,), jnp.float32),
                       pltpu.VMEM((RPW + L,), jnp.float32),
                       pltpu.VMEM((H,), jnp.float32),
                       pltpu.VMEM((H,), jnp.float32),
                       pltpu.VMEM((RPW + L,), jnp.int32),
                       pltpu.VMEM((L,), jnp.int32),
                       pltpu.SemaphoreType.DMA],
        compiler_params=pltpu.CompilerParams(needs_layout_passes=False),
    )
    def _agg(heads_hbm, ls_hbm, ld_hbm, cnt_hbm, hs_hbm, a0_hbm, dinv_hbm,
             b1_hbm, sp_hbm, acc, cntv, headsv, lsx, ldx, gsrc, gdst, rows,
             a0v, dinvv, b1v, sacc, slist, gidx2, sem):
        c = lax.axis_index("c")
        s = lax.axis_index("s")
        w = s * NC + c
        mybase = w * RPW

        zf = jnp.zeros((L,), jnp.float32)
        zi = jnp.zeros((L,), jnp.int32)

        cdesc = pltpu.async_copy(cnt_hbm, cntv, sem)
        adesc = pltpu.async_copy(a0_hbm.at[pl.ds(mybase, RPW)],
                                 a0v.at[pl.ds(0, RPW)], sem)
        ddesc = pltpu.async_copy(dinv_hbm.at[pl.ds(mybase, RPW)],
                                 dinvv.at[pl.ds(0, RPW)], sem)
        bdesc = pltpu.async_copy(b1_hbm, b1v, sem)

        def issue(b):
            ds_ = []
            for li in range(b * LB, (b + 1) * LB):
                ds_.append(pltpu.async_copy(
                    heads_hbm.at[li], headsv.at[li], sem))
            return ds_

        batch = issue(0)

        def pfg(i, carry):
            gsrc[pl.ds(i * L, L)] = zi
            return carry

        lax.fori_loop(0, PCAP // L, pfg, 0)

        def pfs(i, carry):
            slist[pl.ds(i * L, L)] = zi
            return carry

        lax.fori_loop(0, (RPW + L) // L, pfs, 0)
        adesc.wait()
        ddesc.wait()
        bdesc.wait()

        def scomp(g, ns):
            a16 = a0v[pl.ds(g * L, L)]
            m = a16 != 0.0
            idx = lax.iota(jnp.int32, L) + g * L
            plsc.store_compressed(slist.at[pl.ds(ns, L)], idx, mask=m)
            pc = plsc.all_reduce_population_count(m)
            return ns + pc[0]

        ns = lax.fori_loop(0, RPW // L, scomp, jnp.int32(0))

        def zrow(e, carry):
            d = slist[pl.ds(e, L)][0]
            for t in range(H // L):
                acc[d, pl.ds(t * L, L)] = zf
            return carry

        lax.fori_loop(0, ns, zrow, 0)
        cdesc.wait()

        def compact_block(nent, ls_fn, ld_fn, k):
            ngrp = (nent + L - 1) // L

            def comp(g, kk):
                d16 = ld_fn(g)
                s16 = ls_fn(g)
                dl = d16 - mybase
                m = (dl >= 0) & (dl < RPW)
                plsc.store_compressed(gsrc.at[pl.ds(kk, L)], s16, mask=m)
                plsc.store_compressed(gdst.at[pl.ds(kk, L)], dl, mask=m)
                pc = plsc.all_reduce_population_count(m)
                return kk + pc[0]

            return lax.fori_loop(0, ngrp, comp, k)

        def flush(k):
            def gb(j, c3):
                pltpu.sync_copy(hs_hbm.at[gsrc.at[pl.ds(j * L, L)]], rows)

                def lane_body(e, c4):
                    d = gdst[pl.ds(e, L)][0]
                    lane = e - j * L
                    for t in range(H // L):
                        sl = pl.ds(t * L, L)
                        acc[d, sl] += rows[lane, sl]
                    return c4

                lax.fori_loop(j * L, jnp.minimum((j + 1) * L, k), lane_body,
                              0)
                return c3

            lax.fori_loop(0, (k + L - 1) // L, gb, 0)

        def maybe_flush(k):
            @pl.when(k >= FLUSH)
            def _():
                flush(k)
            return jnp.where(k >= FLUSH, 0, k)

        k = jnp.int32(0)
        for b in range(NW // LB):
            for d in batch:
                d.wait()
            if b + 1 < NW // LB:
                batch = issue(b + 1)

            def head_body(li, kk):
                cnt = cntv[li, pl.ds(0, L)][0]
                nent = jnp.minimum(cnt, HEAD)
                kk = compact_block(
                    nent,
                    lambda g: headsv[li, 0, pl.ds(g * L, L)],
                    lambda g: headsv[li, 1, pl.ds(g * L, L)], kk)
                return maybe_flush(kk)

            k = lax.fori_loop(b * LB, (b + 1) * LB, head_body, k)

        def ovf_body(li, kk):
            cnt = cntv[li, pl.ds(0, L)][0]

            def sub_body(sub, k2):
                off = HEAD + sub * HEAD

                def do(kx):
                    pltpu.sync_copy(ls_hbm.at[li, pl.ds(off, HEAD)], lsx)
                    pltpu.sync_copy(ld_hbm.at[li, pl.ds(off, HEAD)], ldx)
                    return compact_block(
                        jnp.minimum(cnt - off, HEAD),
                        lambda g: lsx[pl.ds(g * L, L)],
                        lambda g: ldx[pl.ds(g * L, L)], kx)

                k2 = lax.cond(off < cnt, do, lambda kx: kx, k2)
                return maybe_flush(k2)

            return lax.cond(
                cnt > HEAD,
                lambda kx: lax.fori_loop(0, (LCAP - HEAD) // HEAD, sub_body,
                                         kx),
                lambda kx: kx, kk)

        k = lax.fori_loop(0, NW, ovf_body, k)

        @pl.when(k > 0)
        def _():
            flush(k)

        for t in range(H // L):
            sacc[pl.ds(t * L, L)] = zf

        def rd(j, carry):
            sl16 = slist[pl.ds(j * L, L)]
            gidx2[pl.ds(0, L)] = sl16 + mybase
            pltpu.sync_copy(hs_hbm.at[gidx2], rows)

            def lane_body(e, c4):
                d = slist[pl.ds(e, L)][0]
                lane = e - j * L
                av = a0v[pl.ds(d, L)][0]
                dv = dinvv[pl.ds(d, L)][0]
                for t in range(H // L):
                    sl = pl.ds(t * L, L)
                    h1 = jnp.maximum(
                        dv * (acc[d, sl] + rows[lane, sl]) + b1v[sl], 0.0)
                    sacc[sl] += av * h1
                return c4

            lax.fori_loop(j * L, jnp.minimum((j + 1) * L, ns), lane_body, 0)
            return carry

        lax.fori_loop(0, (ns + L - 1) // L, rd, 0)
        pltpu.sync_copy(sacc, sp_hbm.at[w])

    return _agg


def _final_body(sp_ref, w2_ref, b2_ref, out_ref):
    s = jnp.sum(sp_ref[...], axis=0, keepdims=True)
    out_ref[...] = (jnp.dot(s, w2_ref[...], preferred_element_type=jnp.float32)
                    + b2_ref[...][None, :])


def _final(sp, W2, b2):
    O = W2.shape[1]
    return pl.pallas_call(
        _final_body,
        out_shape=jax.ShapeDtypeStruct((1, O), jnp.float32),
    )(sp, W2, b2)


def kernel(x, edge_index, W1, b1, W2, b2):
    assert x.shape == (N, W1.shape[0]) and edge_index.shape == (2, E)
    src = edge_index[0]
    dst = edge_index[1]
    xp = jnp.pad(x, ((0, NPAD - N), (0, 0)))
    degp, c0p = _hist(src, dst)
    hs, dinv, a0, needed = _mm(xp, W1, degp, c0p)
    heads, ls, ld, cnt = _filt(src, dst, needed.reshape(NPAD))
    sp = _make_agg(W1.shape[1])(heads, ls, ld, cnt, hs, a0.reshape(NPAD),
                                dinv.reshape(NPAD), b1)
    out = _final(sp, W2, b2)
    return out.reshape(W2.shape[1])


# speedup vs baseline: 1.0636x; 1.0636x over previous
import functools

import jax
import jax.numpy as jnp
from jax import lax
from jax.experimental import pallas as pl
from jax.experimental.pallas import tpu as pltpu
from jax.experimental.pallas import tpu_sc as plsc

NC, NS, L = 2, 16, 16
NW = NC * NS
N = 10000
E = 320000
NPAD = 10240
EPW = E // NW
LCAP = NPAD
HEAD = 512
RPW = NPAD // NW
LB = 8
FLUSH = 2048
PCAP = FLUSH + HEAD + L

_mesh = plsc.VectorSubcoreMesh(core_axis_name="c", subcore_axis_name="s")


@functools.partial(
    pl.kernel,
    out_type=(jax.ShapeDtypeStruct((NW, NPAD), jnp.float32),
              jax.ShapeDtypeStruct((NW, NPAD), jnp.float32)),
    mesh=_mesh,
    scratch_types=[pltpu.VMEM((EPW,), jnp.int32),
                   pltpu.VMEM((EPW,), jnp.int32),
                   pltpu.VMEM((NPAD,), jnp.float32),
                   pltpu.VMEM((NPAD,), jnp.float32),
                   pltpu.SemaphoreType.DMA],
    compiler_params=pltpu.CompilerParams(needs_layout_passes=False),
)
def _hist(src_hbm, dst_hbm, degp_hbm, c0p_hbm, srcv, dstv, hdeg, hc0, sem):
    c = lax.axis_index("c")
    s = lax.axis_index("s")
    w = s * NC + c
    d1 = pltpu.async_copy(src_hbm.at[pl.ds(w * EPW, EPW)], srcv, sem)
    d2 = pltpu.async_copy(dst_hbm.at[pl.ds(w * EPW, EPW)], dstv, sem)
    zero = jnp.zeros((L,), jnp.float32)

    def zbody(i, carry):
        hdeg[pl.ds(i * L, L)] = zero
        hc0[pl.ds(i * L, L)] = zero
        return carry

    lax.fori_loop(0, NPAD // L, zbody, 0)
    ones = jnp.ones((L,), jnp.float32)
    d1.wait()
    d2.wait()

    def body(i, carry):
        d16 = dstv[pl.ds(i * L, L)]
        s16 = srcv[pl.ds(i * L, L)]
        plsc.addupdate_scatter(hdeg, [d16], ones)
        plsc.addupdate_scatter(hc0, [s16], ones, mask=d16 == 0)
        return carry

    lax.fori_loop(0, EPW // L, body, 0)
    d3 = pltpu.async_copy(hdeg, degp_hbm.at[w], sem)
    d4 = pltpu.async_copy(hc0, c0p_hbm.at[w], sem)
    d3.wait()
    d4.wait()


def _prep_body(degp_ref, c0p_ref, dinv_ref, a0_ref, needed_ref):
    deg = jnp.sum(degp_ref[...], axis=0, keepdims=True) + 1.0
    dinv = lax.rsqrt(deg)
    cnt0 = jnp.sum(c0p_ref[...], axis=0, keepdims=True)
    col = lax.broadcasted_iota(jnp.int32, (1, NPAD), 1)
    dinv0 = dinv[0, 0]
    a0 = dinv0 * dinv * cnt0 + jnp.where(col == 0, dinv0 * dinv0, 0.0)
    needed = jnp.where((cnt0 > 0.0) | (col == 0), 1.0, 0.0)
    dinv_ref[...] = dinv
    a0_ref[...] = a0
    needed_ref[...] = needed


def _prep(degp, c0p):
    return pl.pallas_call(
        _prep_body,
        out_shape=(jax.ShapeDtypeStruct((1, NPAD), jnp.float32),
                   jax.ShapeDtypeStruct((1, NPAD), jnp.float32),
                   jax.ShapeDtypeStruct((1, NPAD), jnp.float32)),
    )(degp, c0p)


def _mm_body(x_ref, w_ref, dinv_ref, hs_ref):
    h = jnp.dot(x_ref[...], w_ref[...], preferred_element_type=jnp.float32)
    hs_ref[...] = h * dinv_ref[0][:, None]


def _mm(xp, W1, dinv):
    mblk = NPAD // 8
    return pl.pallas_call(
        _mm_body,
        grid=(8,),
        in_specs=[
            pl.BlockSpec((mblk, xp.shape[1]), lambda g: (g, 0)),
            pl.BlockSpec(W1.shape, lambda g: (0, 0)),
            pl.BlockSpec((1, mblk), lambda g: (0, g)),
        ],
        out_specs=pl.BlockSpec((mblk, W1.shape[1]), lambda g: (g, 0)),
        out_shape=jax.ShapeDtypeStruct((NPAD, W1.shape[1]), jnp.float32),
    )(xp, W1, dinv)


@functools.partial(
    pl.kernel,
    out_type=(jax.ShapeDtypeStruct((NW, 2, HEAD), jnp.int32),
              jax.ShapeDtypeStruct((NW, LCAP), jnp.int32),
              jax.ShapeDtypeStruct((NW, LCAP), jnp.int32),
              jax.ShapeDtypeStruct((NW, L), jnp.int32)),
    mesh=_mesh,
    scratch_types=[pltpu.VMEM((EPW,), jnp.int32),
                   pltpu.VMEM((EPW,), jnp.int32),
                   pltpu.VMEM((NPAD,), jnp.float32),
                   pltpu.VMEM((LCAP,), jnp.int32),
                   pltpu.VMEM((LCAP,), jnp.int32),
                   pltpu.VMEM((L,), jnp.int32),
                   pltpu.SemaphoreType.DMA],
    compiler_params=pltpu.CompilerParams(needs_layout_passes=False),
)
def _filt(src_hbm, dst_hbm, needed_hbm, heads_hbm, ls_hbm, ld_hbm, cnt_hbm,
          srcv, dstv, neededv, psrc, pdst, kv, sem):
    c = lax.axis_index("c")
    s = lax.axis_index("s")
    w = s * NC + c
    d1 = pltpu.async_copy(src_hbm.at[pl.ds(w * EPW, EPW)], srcv, sem)
    d2 = pltpu.async_copy(dst_hbm.at[pl.ds(w * EPW, EPW)], dstv, sem)
    d3 = pltpu.async_copy(needed_hbm, neededv, sem)

    zi = jnp.zeros((L,), jnp.int32)
    dummy = jnp.full((L,), NPAD, jnp.int32)

    def pf(i, carry):
        psrc[pl.ds(i * L, L)] = zi
        pdst[pl.ds(i * L, L)] = dummy
        return carry

    lax.fori_loop(0, LCAP // L, pf, 0)
    d1.wait()
    d2.wait()
    d3.wait()

    def comp(i, k):
        d16 = dstv[pl.ds(i * L, L)]
        s16 = srcv[pl.ds(i * L, L)]
        nd = plsc.load_gather(neededv, [d16])
        m = nd > 0.0
        plsc.store_compressed(psrc.at[pl.ds(k, L)], s16, mask=m)
        plsc.store_compressed(pdst.at[pl.ds(k, L)], d16, mask=m)
        pc = plsc.all_reduce_population_count(m)
        return k + pc[0]

    k = lax.fori_loop(0, EPW // L, comp, jnp.int32(0))

    kv[pl.ds(0, L)] = jnp.broadcast_to(k, (L,)).astype(jnp.int32)
    d4 = pltpu.async_copy(psrc.at[pl.ds(0, HEAD)], heads_hbm.at[w, 0], sem)
    d5 = pltpu.async_copy(pdst.at[pl.ds(0, HEAD)], heads_hbm.at[w, 1], sem)
    d6 = pltpu.async_copy(kv, cnt_hbm.at[w], sem)

    @pl.when(k > HEAD)
    def _():
        pltpu.sync_copy(psrc, ls_hbm.at[w])
        pltpu.sync_copy(pdst, ld_hbm.at[w])

    d4.wait()
    d5.wait()
    d6.wait()


def _make_agg(H):
    @functools.partial(
        pl.kernel,
        out_type=jax.ShapeDtypeStruct((NW, H), jnp.float32),
        mesh=_mesh,
        scratch_types=[pltpu.VMEM((RPW, H), jnp.float32),
                       pltpu.VMEM((NW, L), jnp.int32),
                       pltpu.VMEM((NW, 2, HEAD), jnp.int32),
                       pltpu.VMEM((HEAD,), jnp.int32),
                       pltpu.VMEM((HEAD,), jnp.int32),
                       pltpu.VMEM((PCAP,), jnp.int32),
                       pltpu.VMEM((PCAP,), jnp.int32),
                       pltpu.VMEM((L, H), jnp.float32),
                       pltpu.VMEM((RPW + L,), jnp.float32),
                       pltpu.VMEM((RPW + L,), jnp.float32),
                       pltpu.VMEM((H,), jnp.float32),
                       pltpu.VMEM((H,), jnp.float32),
                       pltpu.VMEM((RPW + L,), jnp.int32),
                       pltpu.VMEM((L,), jnp.int32),
                       pltpu.SemaphoreType.DMA],
        compiler_params=pltpu.CompilerParams(needs_layout_passes=False),
    )
    def _agg(heads_hbm, ls_hbm, ld_hbm, cnt_hbm, hs_hbm, a0_hbm, dinv_hbm,
             b1_hbm, sp_hbm, acc, cntv, headsv, lsx, ldx, gsrc, gdst, rows,
             a0v, dinvv, b1v, sacc, slist, gidx2, sem):
        c = lax.axis_index("c")
        s = lax.axis_index("s")
        w = s * NC + c
        mybase = w * RPW

        zf = jnp.zeros((L,), jnp.float32)
        zi = jnp.zeros((L,), jnp.int32)

        cdesc = pltpu.async_copy(cnt_hbm, cntv, sem)
        adesc = pltpu.async_copy(a0_hbm.at[pl.ds(mybase, RPW)],
                                 a0v.at[pl.ds(0, RPW)], sem)
        ddesc = pltpu.async_copy(dinv_hbm.at[pl.ds(mybase, RPW)],
                                 dinvv.at[pl.ds(0, RPW)], sem)
        bdesc = pltpu.async_copy(b1_hbm, b1v, sem)

        def issue(b):
            ds_ = []
            for li in range(b * LB, (b + 1) * LB):
                ds_.append(pltpu.async_copy(
                    heads_hbm.at[li], headsv.at[li], sem))
            return ds_

        batch = issue(0)

        def pfg(i, carry):
            gsrc[pl.ds(i * L, L)] = zi
            return carry

        lax.fori_loop(0, PCAP // L, pfg, 0)

        def pfs(i, carry):
            slist[pl.ds(i * L, L)] = zi
            return carry

        lax.fori_loop(0, (RPW + L) // L, pfs, 0)
        adesc.wait()
        ddesc.wait()
        bdesc.wait()

        def scomp(g, ns):
            a16 = a0v[pl.ds(g * L, L)]
            m = a16 != 0.0
            idx = lax.iota(jnp.int32, L) + g * L
            plsc.store_compressed(slist.at[pl.ds(ns, L)], idx, mask=m)
            pc = plsc.all_reduce_population_count(m)
            return ns + pc[0]

        ns = lax.fori_loop(0, RPW // L, scomp, jnp.int32(0))

        def zrow(e, carry):
            d = slist[pl.ds(e, L)][0]
            for t in range(H // L):
                acc[d, pl.ds(t * L, L)] = zf
            return carry

        lax.fori_loop(0, ns, zrow, 0)
        cdesc.wait()

        def compact_block(nent, ls_fn, ld_fn, k):
            ngrp = (nent + L - 1) // L

            def comp(g, kk):
                d16 = ld_fn(g)
                s16 = ls_fn(g)
                dl = d16 - mybase
                m = (dl >= 0) & (dl < RPW)
                plsc.store_compressed(gsrc.at[pl.ds(kk, L)], s16, mask=m)
                plsc.store_compressed(gdst.at[pl.ds(kk, L)], dl, mask=m)
                pc = plsc.all_reduce_population_count(m)
                return kk + pc[0]

            return lax.fori_loop(0, ngrp, comp, k)

        def flush(k):
            def gb(j, c3):
                pltpu.sync_copy(hs_hbm.at[gsrc.at[pl.ds(j * L, L)]], rows)

                def lane_body(e, c4):
                    d = gdst[pl.ds(e, L)][0]
                    lane = e - j * L
                    for t in range(H // L):
                        sl = pl.ds(t * L, L)
                        acc[d, sl] += rows[lane, sl]
                    return c4

                lax.fori_loop(j * L, jnp.minimum((j + 1) * L, k), lane_body,
                              0)
                return c3

            lax.fori_loop(0, (k + L - 1) // L, gb, 0)

        def maybe_flush(k):
            @pl.when(k >= FLUSH)
            def _():
                flush(k)
            return jnp.where(k >= FLUSH, 0, k)

        k = jnp.int32(0)
        for b in range(NW // LB):
            for d in batch:
                d.wait()
            if b + 1 < NW // LB:
                batch = issue(b + 1)

            def head_body(li, kk):
                cnt = cntv[li, pl.ds(0, L)][0]
                nent = jnp.minimum(cnt, HEAD)
                kk = compact_block(
                    nent,
                    lambda g: headsv[li, 0, pl.ds(g * L, L)],
                    lambda g: headsv[li, 1, pl.ds(g * L, L)], kk)
                return maybe_flush(kk)

            k = lax.fori_loop(b * LB, (b + 1) * LB, head_body, k)

        def ovf_body(li, kk):
            cnt = cntv[li, pl.ds(0, L)][0]

            def sub_body(sub, k2):
                off = HEAD + sub * HEAD

                def do(kx):
                    pltpu.sync_copy(ls_hbm.at[li, pl.ds(off, HEAD)], lsx)
                    pltpu.sync_copy(ld_hbm.at[li, pl.ds(off, HEAD)], ldx)
                    return compact_block(
                        jnp.minimum(cnt - off, HEAD),
                        lambda g: lsx[pl.ds(g * L, L)],
                        lambda g: ldx[pl.ds(g * L, L)], kx)

                k2 = lax.cond(off < cnt, do, lambda kx: kx, k2)
                return maybe_flush(k2)

            return lax.cond(
                cnt > HEAD,
                lambda kx: lax.fori_loop(0, (LCAP - HEAD) // HEAD, sub_body,
                                         kx),
                lambda kx: kx, kk)

        k = lax.fori_loop(0, NW, ovf_body, k)

        @pl.when(k > 0)
        def _():
            flush(k)

        for t in range(H // L):
            sacc[pl.ds(t * L, L)] = zf

        def rd(j, carry):
            sl16 = slist[pl.ds(j * L, L)]
            gidx2[pl.ds(0, L)] = sl16 + mybase
            pltpu.sync_copy(hs_hbm.at[gidx2], rows)

            def lane_body(e, c4):
                d = slist[pl.ds(e, L)][0]
                lane = e - j * L
                av = a0v[pl.ds(d, L)][0]
                dv = dinvv[pl.ds(d, L)][0]
                for t in range(H // L):
                    sl = pl.ds(t * L, L)
                    h1 = jnp.maximum(
                        dv * (acc[d, sl] + rows[lane, sl]) + b1v[sl], 0.0)
                    sacc[sl] += av * h1
                return c4

            lax.fori_loop(j * L, jnp.minimum((j + 1) * L, ns), lane_body, 0)
            return carry

        lax.fori_loop(0, (ns + L - 1) // L, rd, 0)
        pltpu.sync_copy(sacc, sp_hbm.at[w])

    return _agg


def _final_body(sp_ref, w2_ref, b2_ref, out_ref):
    s = jnp.sum(sp_ref[...], axis=0, keepdims=True)
    out_ref[...] = (jnp.dot(s, w2_ref[...], preferred_element_type=jnp.float32)
                    + b2_ref[...][None, :])


def _final(sp, W2, b2):
    O = W2.shape[1]
    return pl.pallas_call(
        _final_body,
        out_shape=jax.ShapeDtypeStruct((1, O), jnp.float32),
    )(sp, W2, b2)


def kernel(x, edge_index, W1, b1, W2, b2):
    assert x.shape == (N, W1.shape[0]) and edge_index.shape == (2, E)
    src = edge_index[0]
    dst = edge_index[1]
    xp = jnp.pad(x, ((0, NPAD - N), (0, 0)))
    degp, c0p = _hist(src, dst)
    dinv, a0, needed = _prep(degp, c0p)
    hs = _mm(xp, W1, dinv)
    heads, ls, ld, cnt = _filt(src, dst, needed.reshape(NPAD))
    sp = _make_agg(W1.shape[1])(heads, ls, ld, cnt, hs, a0.reshape(NPAD),
                                dinv.reshape(NPAD), b1)
    out = _final(sp, W2, b2)
    return out.reshape(W2.shape[1])


# speedup vs baseline: 1.0737x; 1.0095x over previous
import functools

import jax
import jax.numpy as jnp
from jax import lax
from jax.experimental import pallas as pl
from jax.experimental.pallas import tpu as pltpu
from jax.experimental.pallas import tpu_sc as plsc

NC, NS, L = 2, 16, 16
NW = NC * NS
N = 10000
E = 320000
NPAD = 10240
EPW = E // NW
LCAP = NPAD
HEAD = 512
RPW = NPAD // NW
LB = 8
FLUSH = 2048
PCAP = FLUSH + HEAD + L

_mesh = plsc.VectorSubcoreMesh(core_axis_name="c", subcore_axis_name="s")


@functools.partial(
    pl.kernel,
    out_type=(jax.ShapeDtypeStruct((NW, NPAD), jnp.float32),
              jax.ShapeDtypeStruct((NW, NPAD), jnp.float32)),
    mesh=_mesh,
    scratch_types=[pltpu.VMEM((EPW,), jnp.int32),
                   pltpu.VMEM((EPW,), jnp.int32),
                   pltpu.VMEM((NPAD,), jnp.float32),
                   pltpu.VMEM((NPAD,), jnp.float32),
                   pltpu.SemaphoreType.DMA],
    compiler_params=pltpu.CompilerParams(needs_layout_passes=False),
)
def _hist(src_hbm, dst_hbm, degp_hbm, c0p_hbm, srcv, dstv, hdeg, hc0, sem):
    c = lax.axis_index("c")
    s = lax.axis_index("s")
    w = s * NC + c
    d1 = pltpu.async_copy(src_hbm.at[pl.ds(w * EPW, EPW)], srcv, sem)
    d2 = pltpu.async_copy(dst_hbm.at[pl.ds(w * EPW, EPW)], dstv, sem)
    zero = jnp.zeros((L,), jnp.float32)

    def zbody(i, carry):
        hdeg[pl.ds(i * L, L)] = zero
        hc0[pl.ds(i * L, L)] = zero
        return carry

    lax.fori_loop(0, NPAD // L, zbody, 0)
    ones = jnp.ones((L,), jnp.float32)
    d1.wait()
    d2.wait()

    def body(i, carry):
        d16 = dstv[pl.ds(i * L, L)]
        s16 = srcv[pl.ds(i * L, L)]
        plsc.addupdate_scatter(hdeg, [d16], ones)
        plsc.addupdate_scatter(hc0, [s16], ones, mask=d16 == 0)
        return carry

    lax.fori_loop(0, EPW // L, body, 0)
    d3 = pltpu.async_copy(hdeg, degp_hbm.at[w], sem)
    d4 = pltpu.async_copy(hc0, c0p_hbm.at[w], sem)
    d3.wait()
    d4.wait()


def _prep_body(degp_ref, c0p_ref, dinv_ref, a0_ref, needed_ref):
    deg = jnp.sum(degp_ref[...], axis=0, keepdims=True) + 1.0
    dinv = lax.rsqrt(deg)
    cnt0 = jnp.sum(c0p_ref[...], axis=0, keepdims=True)
    col = lax.broadcasted_iota(jnp.int32, (1, NPAD), 1)
    dinv0 = dinv[0, 0]
    a0 = dinv0 * dinv * cnt0 + jnp.where(col == 0, dinv0 * dinv0, 0.0)
    needed = jnp.where((cnt0 > 0.0) | (col == 0), 1.0, 0.0)
    dinv_ref[...] = dinv
    a0_ref[...] = a0
    needed_ref[...] = needed


def _prep(degp, c0p):
    return pl.pallas_call(
        _prep_body,
        out_shape=(jax.ShapeDtypeStruct((1, NPAD), jnp.float32),
                   jax.ShapeDtypeStruct((1, NPAD), jnp.float32),
                   jax.ShapeDtypeStruct((1, NPAD), jnp.float32)),
    )(degp, c0p)


def _mm_body(x_ref, w_ref, dinv_ref, hs_ref):
    h = jnp.dot(x_ref[...], w_ref[...], preferred_element_type=jnp.float32)
    hs_ref[...] = h * dinv_ref[0][:, None]


def _mm(xp, W1, dinv):
    mblk = NPAD // 8
    return pl.pallas_call(
        _mm_body,
        grid=(8,),
        in_specs=[
            pl.BlockSpec((mblk, xp.shape[1]), lambda g: (g, 0)),
            pl.BlockSpec(W1.shape, lambda g: (0, 0)),
            pl.BlockSpec((1, mblk), lambda g: (0, g)),
        ],
        out_specs=pl.BlockSpec((mblk, W1.shape[1]), lambda g: (g, 0)),
        out_shape=jax.ShapeDtypeStruct((NPAD, W1.shape[1]), jnp.float32),
    )(xp, W1, dinv)


@functools.partial(
    pl.kernel,
    out_type=(jax.ShapeDtypeStruct((NW, 2, HEAD), jnp.int32),
              jax.ShapeDtypeStruct((NW, LCAP), jnp.int32),
              jax.ShapeDtypeStruct((NW, LCAP), jnp.int32),
              jax.ShapeDtypeStruct((NW, L), jnp.int32)),
    mesh=_mesh,
    scratch_types=[pltpu.VMEM((EPW,), jnp.int32),
                   pltpu.VMEM((EPW,), jnp.int32),
                   pltpu.VMEM((NPAD,), jnp.float32),
                   pltpu.VMEM((LCAP,), jnp.int32),
                   pltpu.VMEM((LCAP,), jnp.int32),
                   pltpu.VMEM((L,), jnp.int32),
                   pltpu.SemaphoreType.DMA],
    compiler_params=pltpu.CompilerParams(needs_layout_passes=False),
)
def _filt(src_hbm, dst_hbm, needed_hbm, heads_hbm, ls_hbm, ld_hbm, cnt_hbm,
          srcv, dstv, neededv, psrc, pdst, kv, sem):
    c = lax.axis_index("c")
    s = lax.axis_index("s")
    w = s * NC + c
    d1 = pltpu.async_copy(src_hbm.at[pl.ds(w * EPW, EPW)], srcv, sem)
    d2 = pltpu.async_copy(dst_hbm.at[pl.ds(w * EPW, EPW)], dstv, sem)
    d3 = pltpu.async_copy(needed_hbm, neededv, sem)

    zi = jnp.zeros((L,), jnp.int32)
    dummy = jnp.full((L,), NPAD, jnp.int32)

    def pf(i, carry):
        psrc[pl.ds(i * L, L)] = zi
        pdst[pl.ds(i * L, L)] = dummy
        return carry

    lax.fori_loop(0, LCAP // L, pf, 0)
    d1.wait()
    d2.wait()
    d3.wait()

    def comp(i, k):
        d16 = dstv[pl.ds(i * L, L)]
        s16 = srcv[pl.ds(i * L, L)]
        nd = plsc.load_gather(neededv, [d16])
        m = nd > 0.0
        plsc.store_compressed(psrc.at[pl.ds(k, L)], s16, mask=m)
        plsc.store_compressed(pdst.at[pl.ds(k, L)], d16, mask=m)
        pc = plsc.all_reduce_population_count(m)
        return k + pc[0]

    k = lax.fori_loop(0, EPW // L, comp, jnp.int32(0))

    kv[pl.ds(0, L)] = jnp.broadcast_to(k, (L,)).astype(jnp.int32)
    d4 = pltpu.async_copy(psrc.at[pl.ds(0, HEAD)], heads_hbm.at[w, 0], sem)
    d5 = pltpu.async_copy(pdst.at[pl.ds(0, HEAD)], heads_hbm.at[w, 1], sem)
    d6 = pltpu.async_copy(kv, cnt_hbm.at[w], sem)

    @pl.when(k > HEAD)
    def _():
        pltpu.sync_copy(psrc, ls_hbm.at[w])
        pltpu.sync_copy(pdst, ld_hbm.at[w])

    d4.wait()
    d5.wait()
    d6.wait()


def _make_agg(H):
    @functools.partial(
        pl.kernel,
        out_type=jax.ShapeDtypeStruct((NW, H), jnp.float32),
        mesh=_mesh,
        scratch_types=[pltpu.VMEM((RPW, H), jnp.float32),
                       pltpu.VMEM((NW, L), jnp.int32),
                       pltpu.VMEM((NW, 2, HEAD), jnp.int32),
                       pltpu.VMEM((HEAD,), jnp.int32),
                       pltpu.VMEM((HEAD,), jnp.int32),
                       pltpu.VMEM((PCAP,), jnp.int32),
                       pltpu.VMEM((PCAP,), jnp.int32),
                       pltpu.VMEM((L, H), jnp.float32),
                       pltpu.VMEM((RPW + L,), jnp.float32),
                       pltpu.VMEM((RPW + L,), jnp.float32),
                       pltpu.VMEM((H,), jnp.float32),
                       pltpu.VMEM((H,), jnp.float32),
                       pltpu.VMEM((RPW + L,), jnp.int32),
                       pltpu.VMEM((L,), jnp.int32),
                       pltpu.SemaphoreType.DMA],
        compiler_params=pltpu.CompilerParams(needs_layout_passes=False),
    )
    def _agg(heads_hbm, ls_hbm, ld_hbm, cnt_hbm, hs_hbm, a0_hbm, dinv_hbm,
             b1_hbm, sp_hbm, acc, cntv, headsv, lsx, ldx, gsrc, gdst, rows,
             a0v, dinvv, b1v, sacc, slist, gidx2, sem):
        c = lax.axis_index("c")
        s = lax.axis_index("s")
        w = s * NC + c
        mybase = w * RPW

        zf = jnp.zeros((L,), jnp.float32)
        zi = jnp.zeros((L,), jnp.int32)

        cdesc = pltpu.async_copy(cnt_hbm, cntv, sem)
        adesc = pltpu.async_copy(a0_hbm.at[pl.ds(mybase, RPW)],
                                 a0v.at[pl.ds(0, RPW)], sem)
        ddesc = pltpu.async_copy(dinv_hbm.at[pl.ds(mybase, RPW)],
                                 dinvv.at[pl.ds(0, RPW)], sem)
        bdesc = pltpu.async_copy(b1_hbm, b1v, sem)

        def issue(b):
            ds_ = []
            for li in range(b * LB, (b + 1) * LB):
                ds_.append(pltpu.async_copy(
                    heads_hbm.at[li], headsv.at[li], sem))
            return ds_

        batch = issue(0)

        def pfg(i, carry):
            gsrc[pl.ds(i * L, L)] = zi
            return carry

        lax.fori_loop(0, PCAP // L, pfg, 0)

        def pfs(i, carry):
            slist[pl.ds(i * L, L)] = zi
            return carry

        lax.fori_loop(0, (RPW + L) // L, pfs, 0)
        adesc.wait()
        ddesc.wait()
        bdesc.wait()

        def scomp(g, ns):
            a16 = a0v[pl.ds(g * L, L)]
            m = a16 != 0.0
            idx = lax.iota(jnp.int32, L) + g * L
            plsc.store_compressed(slist.at[pl.ds(ns, L)], idx, mask=m)
            pc = plsc.all_reduce_population_count(m)
            return ns + pc[0]

        ns = lax.fori_loop(0, RPW // L, scomp, jnp.int32(0))

        def zrow(e, carry):
            d = slist[pl.ds(e, L)][0]
            for t in range(H // L):
                acc[d, pl.ds(t * L, L)] = zf
            return carry

        lax.fori_loop(0, ns, zrow, 0)
        cdesc.wait()

        def compact_block(nent, ls_fn, ld_fn, k):
            ngrp = (nent + L - 1) // L

            def comp(g, kk):
                d16 = ld_fn(g)
                s16 = ls_fn(g)
                dl = d16 - mybase
                m = (dl >= 0) & (dl < RPW)
                plsc.store_compressed(gsrc.at[pl.ds(kk, L)], s16, mask=m)
                plsc.store_compressed(gdst.at[pl.ds(kk, L)], dl, mask=m)
                pc = plsc.all_reduce_population_count(m)
                return kk + pc[0]

            return lax.fori_loop(0, ngrp, comp, k)

        def flush(k):
            def gb(j, c3):
                pltpu.sync_copy(hs_hbm.at[gsrc.at[pl.ds(j * L, L)]], rows)

                def lane_body(e, c4):
                    d = gdst[pl.ds(e, L)][0]
                    lane = e - j * L
                    for t in range(H // L):
                        sl = pl.ds(t * L, L)
                        acc[d, sl] += rows[lane, sl]
                    return c4

                lax.fori_loop(j * L, jnp.minimum((j + 1) * L, k), lane_body,
                              0)
                return c3

            lax.fori_loop(0, (k + L - 1) // L, gb, 0)

        def maybe_flush(k):
            @pl.when(k >= FLUSH)
            def _():
                flush(k)
            return jnp.where(k >= FLUSH, 0, k)

        k = jnp.int32(0)
        nb_total = NW // LB
        batches = [batch, issue(1)]
        for b in range(nb_total):
            for d in batches[b]:
                d.wait()
            if b + 2 < nb_total:
                batches.append(issue(b + 2))

            def head_body(li, kk):
                cnt = cntv[li, pl.ds(0, L)][0]
                nent = jnp.minimum(cnt, HEAD)
                kk = compact_block(
                    nent,
                    lambda g: headsv[li, 0, pl.ds(g * L, L)],
                    lambda g: headsv[li, 1, pl.ds(g * L, L)], kk)
                return maybe_flush(kk)

            k = lax.fori_loop(b * LB, (b + 1) * LB, head_body, k)

        def ovf_body(li, kk):
            cnt = cntv[li, pl.ds(0, L)][0]

            def sub_body(sub, k2):
                off = HEAD + sub * HEAD

                def do(kx):
                    pltpu.sync_copy(ls_hbm.at[li, pl.ds(off, HEAD)], lsx)
                    pltpu.sync_copy(ld_hbm.at[li, pl.ds(off, HEAD)], ldx)
                    return compact_block(
                        jnp.minimum(cnt - off, HEAD),
                        lambda g: lsx[pl.ds(g * L, L)],
                        lambda g: ldx[pl.ds(g * L, L)], kx)

                k2 = lax.cond(off < cnt, do, lambda kx: kx, k2)
                return maybe_flush(k2)

            return lax.cond(
                cnt > HEAD,
                lambda kx: lax.fori_loop(0, (LCAP - HEAD) // HEAD, sub_body,
                                         kx),
                lambda kx: kx, kk)

        k = lax.fori_loop(0, NW, ovf_body, k)

        @pl.when(k > 0)
        def _():
            flush(k)

        for t in range(H // L):
            sacc[pl.ds(t * L, L)] = zf

        def rd(j, carry):
            sl16 = slist[pl.ds(j * L, L)]
            gidx2[pl.ds(0, L)] = sl16 + mybase
            pltpu.sync_copy(hs_hbm.at[gidx2], rows)

            def lane_body(e, c4):
                d = slist[pl.ds(e, L)][0]
                lane = e - j * L
                av = a0v[pl.ds(d, L)][0]
                dv = dinvv[pl.ds(d, L)][0]
                for t in range(H // L):
                    sl = pl.ds(t * L, L)
                    h1 = jnp.maximum(
                        dv * (acc[d, sl] + rows[lane, sl]) + b1v[sl], 0.0)
                    sacc[sl] += av * h1
                return c4

            lax.fori_loop(j * L, jnp.minimum((j + 1) * L, ns), lane_body, 0)
            return carry

        lax.fori_loop(0, (ns + L - 1) // L, rd, 0)
        pltpu.sync_copy(sacc, sp_hbm.at[w])

    return _agg


def _final_body(sp_ref, w2_ref, b2_ref, out_ref):
    s = jnp.sum(sp_ref[...], axis=0, keepdims=True)
    out_ref[...] = (jnp.dot(s, w2_ref[...], preferred_element_type=jnp.float32)
                    + b2_ref[...][None, :])


def _final(sp, W2, b2):
    O = W2.shape[1]
    return pl.pallas_call(
        _final_body,
        out_shape=jax.ShapeDtypeStruct((1, O), jnp.float32),
    )(sp, W2, b2)


def kernel(x, edge_index, W1, b1, W2, b2):
    assert x.shape == (N, W1.shape[0]) and edge_index.shape == (2, E)
    src = edge_index[0]
    dst = edge_index[1]
    xp = jnp.pad(x, ((0, NPAD - N), (0, 0)))
    degp, c0p = _hist(src, dst)
    dinv, a0, needed = _prep(degp, c0p)
    hs = _mm(xp, W1, dinv)
    heads, ls, ld, cnt = _filt(src, dst, needed.reshape(NPAD))
    sp = _make_agg(W1.shape[1])(heads, ls, ld, cnt, hs, a0.reshape(NPAD),
                                dinv.reshape(NPAD), b1)
    out = _final(sp, W2, b2)
    return out.reshape(W2.shape[1])
